# baseline (speedup 1.0000x reference)
import functools

import jax
import jax.numpy as jnp
from jax import lax
from jax.experimental import pallas as pl
from jax.experimental.pallas import tpu as pltpu
from jax.experimental.pallas import tpu_sc as plsc

B = 16
P = 512
D = 256
T = 2048
L = 16
HALF = T // 2
C = 128
NCH = HALF // C


def _zero_rows(buf, r_lo, r_hi):
    zeros = jnp.zeros((L,), jnp.float32)

    def body(r, _):
        for k in range(D // L):
            buf[r, pl.ds(k * L, L)] = zeros
        return 0

    lax.fori_loop(r_lo, r_hi, body, 0)


def _lr_body(x_hbm, dur_hbm, out_hbm, mel_hbm,
             dur_v, map_v, idx_v, gbuf, mel_v, gsem, wsem0, wsem1):
    b = lax.axis_index("s") ^ 3
    h = lax.axis_index("c")

    lane15 = jnp.full((L,), L - 1, jnp.int32)

    dur_dma = pltpu.async_copy(dur_hbm.at[b], dur_v, gsem)
    zi = jnp.zeros((L,), jnp.int32)

    def zmap(i, _):
        map_v[pl.ds(i * L, L)] = zi
        return 0

    with jax.named_scope("zmap"):
        lax.fori_loop(0, T // L, zmap, 0)
        dur_dma.wait()

    def csum(i, carry):
        d = dur_v[pl.ds(i * L, L)]
        c = plsc.cumsum(d)
        s = c + carry
        start = s - d
        pvec = lax.iota(jnp.int32, 16) + i * L
        msk = (d > 0) & (start < T)
        plsc.store_scatter(map_v, [start], pvec, mask=msk)
        return carry + jnp.take_along_axis(c, lane15, axis=0)

    with jax.named_scope("pass1"):
        mel_vec = lax.fori_loop(0, P // L, csum, jnp.zeros((L,), jnp.int32))
        mel_len = jnp.max(mel_vec)

    base = b * P
    UNR = 4

    def cmx(i, mc):
        for k in range(UNR):
            off = i * (UNR * L) + k * L
            t = plsc.cummax(map_v[pl.ds(off, L)])
            tb = jnp.take_along_axis(t, lane15, axis=0)
            idx_v[pl.ds(off, L)] = jnp.minimum(jnp.maximum(t, mc), P - 1) + base
            mc = jnp.maximum(mc, tb)
        return mc

    with jax.named_scope("pass2"):
        lax.fori_loop(0, T // (UNR * L), cmx, jnp.zeros((L,), jnp.int32))

    nv = jnp.clip(mel_len, 0, T)
    row0 = b * T
    bufs = (gbuf.at[0], gbuf.at[1])
    wsems = (wsem0, wsem1)
    par = h ^ (b & 1)

    def chunk_pair(jj, _):
        for q in range(2):
            g = 4 * jj + 2 * q + par
            nvj = nv - g * C
            zc = jnp.clip(nvj, 0, C)
            prev = jnp.where(jj > 0, jnp.clip(nvj + 4 * C, 0, C), C)

            @pl.when(jj > 0)
            def _wait_prev():
                pltpu.make_async_copy(
                    bufs[q], out_hbm.at[pl.ds(row0 + (g - 4) * C, C)], wsems[q]
                ).wait()

            @pl.when(nvj > 0)
            def _gather():
                idx_slice = idx_v.at[pl.ds(g * C, C)]
                pltpu.async_copy(x_hbm.at[idx_slice], bufs[q], gsem).wait()

            _zero_rows(bufs[q], zc, prev)

            pltpu.async_copy(
                bufs[q], out_hbm.at[pl.ds(row0 + g * C, C)], wsems[q])
        return 0

    with jax.named_scope("chunks"):
        lax.fori_loop(0, NCH // 2, chunk_pair, 0)

    for q in range(2):
        g = 2 * (NCH - 2 + q) + par
        pltpu.make_async_copy(
            bufs[q], out_hbm.at[pl.ds(row0 + g * C, C)], wsems[q]
        ).wait()

    @pl.when(h == 0)
    def _write_mel():
        mel_v[...] = mel_vec
        pltpu.sync_copy(mel_v, mel_hbm.at[b])


@jax.jit
def _length_regulate(x_flat, duration):
    mesh = plsc.VectorSubcoreMesh(core_axis_name="c", subcore_axis_name="s")
    out, mel = pl.kernel(
        _lr_body,
        out_type=[
            jax.ShapeDtypeStruct((B * T, D), jnp.float32),
            jax.ShapeDtypeStruct((B, L), jnp.int32),
        ],
        mesh=mesh,
        compiler_params=pltpu.CompilerParams(needs_layout_passes=False),
        scratch_types=[
            pltpu.VMEM((P,), jnp.int32),
            pltpu.VMEM((T,), jnp.int32),
            pltpu.VMEM((T,), jnp.int32),
            pltpu.VMEM((2, C, D), jnp.float32),
            pltpu.VMEM((L,), jnp.int32),
            pltpu.SemaphoreType.DMA,
            pltpu.SemaphoreType.DMA,
            pltpu.SemaphoreType.DMA,
        ],
    )(x_flat, duration)
    return out, mel


def kernel(x, duration, max_len):
    x_flat = x.reshape(B * P, D)
    out, mel = _length_regulate(x_flat, duration.astype(jnp.int32))
    return out.reshape(B, T, D), mel[:, 0]


# speedup vs baseline: 1.0128x; 1.0128x over previous
import functools

import jax
import jax.numpy as jnp
from jax import lax
from jax.experimental import pallas as pl
from jax.experimental.pallas import tpu as pltpu
from jax.experimental.pallas import tpu_sc as plsc

B = 16
P = 512
D = 256
T = 2048
L = 16
HALF = T // 2
C = 128
NCH = HALF // C


def _zero_rows(buf, r_lo, r_hi):
    zeros = jnp.zeros((L,), jnp.float32)

    def body(r, _):
        for k in range(D // L):
            buf[r, pl.ds(k * L, L)] = zeros
        return 0

    lax.fori_loop(r_lo, r_hi, body, 0)


def _lr_body(x_hbm, dur_hbm, out_hbm, mel_hbm,
             dur_v, map_v, idx_v, gbuf, mel_v, gsem, wsem0, wsem1):
    b = lax.axis_index("s") ^ 3
    h = lax.axis_index("c")

    lane15 = jnp.full((L,), L - 1, jnp.int32)

    dur_dma = pltpu.async_copy(dur_hbm.at[b], dur_v, gsem)
    zi = jnp.zeros((L,), jnp.int32)

    def zmap(i, _):
        map_v[pl.ds(i * L, L)] = zi
        return 0

    with jax.named_scope("zmap"):
        lax.fori_loop(0, T // L, zmap, 0)
        dur_dma.wait()

    def csum(i, carry):
        d = dur_v[pl.ds(i * L, L)]
        c = plsc.cumsum(d)
        s = c + carry
        start = s - d
        pvec = lax.iota(jnp.int32, 16) + i * L
        msk = (d > 0) & (start < T)
        plsc.store_scatter(map_v, [start], pvec, mask=msk)
        return carry + jnp.take_along_axis(c, lane15, axis=0)

    with jax.named_scope("pass1"):
        mel_vec = lax.fori_loop(0, P // L, csum, jnp.zeros((L,), jnp.int32))
        mel_len = jnp.max(mel_vec)

    base = b * P
    UNR = 4

    def cmx(i, mc):
        for k in range(UNR):
            off = i * (UNR * L) + k * L
            t = plsc.cummax(map_v[pl.ds(off, L)])
            tb = jnp.take_along_axis(t, lane15, axis=0)
            idx_v[pl.ds(off, L)] = jnp.minimum(jnp.maximum(t, mc), P - 1) + base
            mc = jnp.maximum(mc, tb)
        return mc

    with jax.named_scope("pass2"):
        lax.fori_loop(0, T // (UNR * L), cmx, jnp.zeros((L,), jnp.int32))

    nv = jnp.clip(mel_len, 0, T)
    row0 = b * T
    bufs = (gbuf.at[0], gbuf.at[1])
    wsems = (wsem0, wsem1)
    par = h ^ (b & 1)

    def chunk_pair(jj, _):
        for q in range(2):
            g = 4 * jj + 2 * q + par
            nvj = nv - g * C
            zc = jnp.clip(nvj, 0, C)
            prev = jnp.where(jj > 0, jnp.clip(nvj + 4 * C, 0, C), C)

            with jax.named_scope("cwait"):
                @pl.when(jj > 0)
                def _wait_prev():
                    pltpu.make_async_copy(
                        bufs[q], out_hbm.at[pl.ds(row0 + (g - 4) * C, C)],
                        wsems[q]
                    ).wait()

            with jax.named_scope("cgather"):
                @pl.when(nvj > 0)
                def _gather():
                    idx_slice = idx_v.at[pl.ds(g * C, C)]
                    pltpu.async_copy(x_hbm.at[idx_slice], bufs[q], gsem).wait()

            with jax.named_scope("czero"):
                _zero_rows(bufs[q], zc, prev)

            pltpu.async_copy(
                bufs[q], out_hbm.at[pl.ds(row0 + g * C, C)], wsems[q])
        return 0

    with jax.named_scope("chunks"):
        lax.fori_loop(0, NCH // 2, chunk_pair, 0)

    for q in range(2):
        g = 2 * (NCH - 2 + q) + par
        pltpu.make_async_copy(
            bufs[q], out_hbm.at[pl.ds(row0 + g * C, C)], wsems[q]
        ).wait()

    @pl.when(h == 0)
    def _write_mel():
        mel_v[...] = mel_vec
        pltpu.sync_copy(mel_v, mel_hbm.at[b])


@jax.jit
def _length_regulate(x_flat, duration):
    mesh = plsc.VectorSubcoreMesh(core_axis_name="c", subcore_axis_name="s")
    out, mel = pl.kernel(
        _lr_body,
        out_type=[
            jax.ShapeDtypeStruct((B * T, D), jnp.float32),
            jax.ShapeDtypeStruct((B, L), jnp.int32),
        ],
        mesh=mesh,
        compiler_params=pltpu.CompilerParams(needs_layout_passes=False),
        scratch_types=[
            pltpu.VMEM((P,), jnp.int32),
            pltpu.VMEM((T,), jnp.int32),
            pltpu.VMEM((T,), jnp.int32),
            pltpu.VMEM((2, C, D), jnp.float32),
            pltpu.VMEM((L,), jnp.int32),
            pltpu.SemaphoreType.DMA,
            pltpu.SemaphoreType.DMA,
            pltpu.SemaphoreType.DMA,
        ],
    )(x_flat, duration)
    return out, mel


def kernel(x, duration, max_len):
    x_flat = x.reshape(B * P, D)
    out, mel = _length_regulate(x_flat, duration.astype(jnp.int32))
    return out.reshape(B, T, D), mel[:, 0]


# speedup vs baseline: 1.2706x; 1.2545x over previous
import functools

import jax
import jax.numpy as jnp
from jax import lax
from jax.experimental import pallas as pl
from jax.experimental.pallas import tpu as pltpu
from jax.experimental.pallas import tpu_sc as plsc

B = 16
P = 512
D = 256
T = 2048
L = 16
HALF = T // 2
C = 128
NCH = HALF // C


def _zero_rows(buf, r_lo, r_hi):
    zeros = jnp.zeros((L,), jnp.float32)

    def body(r, _):
        for k in range(D // L):
            buf[r, pl.ds(k * L, L)] = zeros
        return 0

    lax.fori_loop(r_lo, r_hi, body, 0)


def _lr_body(x_hbm, dur_hbm, out_hbm, mel_hbm,
             dur_v, map_v, idx_v, gbuf, mel_v, gsem, wsem0, wsem1):
    b = lax.axis_index("s")
    h = lax.axis_index("c")

    lane15 = jnp.full((L,), L - 1, jnp.int32)

    dur_dma = pltpu.async_copy(dur_hbm.at[b], dur_v, gsem)
    zi = jnp.zeros((L,), jnp.int32)

    def zmap(i, _):
        map_v[pl.ds(i * L, L)] = zi
        return 0

    lax.fori_loop(0, T // L, zmap, 0)
    dur_dma.wait()

    def csum(i, carry):
        d = dur_v[pl.ds(i * L, L)]
        c = plsc.cumsum(d)
        s = c + carry
        start = s - d
        pvec = lax.iota(jnp.int32, 16) + i * L
        msk = (d > 0) & (start < T)
        plsc.store_scatter(map_v, [start], pvec, mask=msk)
        return carry + jnp.take_along_axis(c, lane15, axis=0)

    mel_vec = lax.fori_loop(0, P // L, csum, jnp.zeros((L,), jnp.int32))
    mel_len = jnp.max(mel_vec)

    base = b * P
    UNR = 4

    def cmx(i, mc):
        for k in range(UNR):
            off = i * (UNR * L) + k * L
            t = plsc.cummax(map_v[pl.ds(off, L)])
            tb = jnp.take_along_axis(t, lane15, axis=0)
            fr = lax.iota(jnp.int32, 16) + off
            ph = jnp.where(fr < mel_vec, jnp.maximum(t, mc), fr & (P - 1))
            idx_v[pl.ds(off, L)] = ph + base
            mc = jnp.maximum(mc, tb)
        return mc

    lax.fori_loop(0, T // (UNR * L), cmx, jnp.zeros((L,), jnp.int32))

    nv = jnp.clip(mel_len, 0, T)
    row0 = b * T
    bufs = (gbuf.at[0], gbuf.at[1])
    wsems = (wsem0, wsem1)
    par = h ^ (b & 1)

    def chunk_pair(jj, _):
        for q in range(2):
            g = 4 * jj + 2 * q + par
            nvj = nv - g * C
            zc = jnp.clip(nvj, 0, C)
            prev = jnp.where(jj > 0, jnp.clip(nvj + 4 * C, 0, C), C)

            @pl.when(jj > 0)
            def _wait_prev():
                pltpu.make_async_copy(
                    bufs[q], out_hbm.at[pl.ds(row0 + (g - 4) * C, C)], wsems[q]
                ).wait()

            @pl.when(nvj > 0)
            def _gather():
                idx_slice = idx_v.at[pl.ds(g * C, C)]
                pltpu.async_copy(x_hbm.at[idx_slice], bufs[q], gsem).wait()

            _zero_rows(bufs[q], zc, prev)

            pltpu.async_copy(
                bufs[q], out_hbm.at[pl.ds(row0 + g * C, C)], wsems[q])
        return 0

    lax.fori_loop(0, NCH // 2, chunk_pair, 0)

    for q in range(2):
        g = 2 * (NCH - 2 + q) + par
        pltpu.make_async_copy(
            bufs[q], out_hbm.at[pl.ds(row0 + g * C, C)], wsems[q]
        ).wait()

    @pl.when(h == 0)
    def _write_mel():
        mel_v[...] = mel_vec
        pltpu.sync_copy(mel_v, mel_hbm.at[b])


@jax.jit
def _length_regulate(x_flat, duration):
    mesh = plsc.VectorSubcoreMesh(core_axis_name="c", subcore_axis_name="s")
    out, mel = pl.kernel(
        _lr_body,
        out_type=[
            jax.ShapeDtypeStruct((B * T, D), jnp.float32),
            jax.ShapeDtypeStruct((B, L), jnp.int32),
        ],
        mesh=mesh,
        compiler_params=pltpu.CompilerParams(needs_layout_passes=False),
        scratch_types=[
            pltpu.VMEM((P,), jnp.int32),
            pltpu.VMEM((T,), jnp.int32),
            pltpu.VMEM((T,), jnp.int32),
            pltpu.VMEM((2, C, D), jnp.float32),
            pltpu.VMEM((L,), jnp.int32),
            pltpu.SemaphoreType.DMA,
            pltpu.SemaphoreType.DMA,
            pltpu.SemaphoreType.DMA,
        ],
    )(x_flat, duration)
    return out, mel


def kernel(x, duration, max_len):
    x_flat = x.reshape(B * P, D)
    out, mel = _length_regulate(x_flat, duration.astype(jnp.int32))
    return out.reshape(B, T, D), mel[:, 0]


# speedup vs baseline: 1.3162x; 1.0359x over previous
import functools

import jax
import jax.numpy as jnp
from jax import lax
from jax.experimental import pallas as pl
from jax.experimental.pallas import tpu as pltpu
from jax.experimental.pallas import tpu_sc as plsc

B = 16
P = 512
D = 256
T = 2048
L = 16
HALF = T // 2
C = 128
NCH = HALF // C


def _zero_rows(buf, r_lo, r_hi):
    zeros = jnp.zeros((L,), jnp.float32)

    def body(r, _):
        for k in range(D // L):
            buf[r, pl.ds(k * L, L)] = zeros
        return 0

    lax.fori_loop(r_lo, r_hi, body, 0)


def _lr_body(x_hbm, dur_hbm, out_hbm, mel_hbm,
             dur_v, map_v, idx_v, gbuf, mel_v, mel2_v, shmel,
             gsem, wsem0, wsem1):
    b = lax.axis_index("s")
    h = lax.axis_index("c")

    lane15 = jnp.full((L,), L - 1, jnp.int32)

    dur_dma = pltpu.async_copy(dur_hbm.at[b], dur_v, gsem)
    zi = jnp.zeros((L,), jnp.int32)

    def zmap(i, _):
        map_v[pl.ds(i * L, L)] = zi
        return 0

    lax.fori_loop(0, T // L, zmap, 0)
    dur_dma.wait()

    def csum(i, carry):
        d = dur_v[pl.ds(i * L, L)]
        c = plsc.cumsum(d)
        s = c + carry
        start = s - d
        pvec = lax.iota(jnp.int32, 16) + i * L
        msk = (d > 0) & (start < T)
        plsc.store_scatter(map_v, [start], pvec, mask=msk)
        return carry + jnp.take_along_axis(c, lane15, axis=0)

    mel_vec = lax.fori_loop(0, P // L, csum, jnp.zeros((L,), jnp.int32))
    mel_len = jnp.max(mel_vec)

    @pl.when(h == 0)
    def _stage_mel():
        mel_v[...] = mel_vec
        pltpu.sync_copy(mel_v, shmel.at[b])

    plsc.subcore_barrier()

    @pl.when((h == 0) & (b == 0))
    def _emit_mel():
        pltpu.sync_copy(shmel, mel2_v)
        vals = plsc.load_gather(
            mel2_v, [lax.iota(jnp.int32, 16), jnp.zeros((L,), jnp.int32)])
        mel_v[...] = vals
        pltpu.sync_copy(mel_v, mel_hbm)

    base = b * P
    UNR = 4

    def cmx(i, mc):
        for k in range(UNR):
            off = i * (UNR * L) + k * L
            t = plsc.cummax(map_v[pl.ds(off, L)])
            tb = jnp.take_along_axis(t, lane15, axis=0)
            fr = lax.iota(jnp.int32, 16) + off
            ph = jnp.where(fr < mel_vec, jnp.maximum(t, mc), fr & (P - 1))
            idx_v[pl.ds(off, L)] = ph + base
            mc = jnp.maximum(mc, tb)
        return mc

    lax.fori_loop(0, T // (UNR * L), cmx, jnp.zeros((L,), jnp.int32))

    nv = jnp.clip(mel_len, 0, T)
    row0 = b * T
    bufs = (gbuf.at[0], gbuf.at[1])
    wsems = (wsem0, wsem1)
    par = h ^ (b & 1)

    def chunk_pair(jj, _):
        for q in range(2):
            g = 4 * jj + 2 * q + par
            nvj = nv - g * C
            zc = jnp.clip(nvj, 0, C)
            prev = jnp.where(jj > 0, jnp.clip(nvj + 4 * C, 0, C), C)

            @pl.when(jj > 0)
            def _wait_prev():
                pltpu.make_async_copy(
                    bufs[q], out_hbm.at[pl.ds(row0 + (g - 4) * C, C)], wsems[q]
                ).wait()

            @pl.when(nvj > 0)
            def _gather():
                idx_slice = idx_v.at[pl.ds(g * C, C)]
                pltpu.async_copy(x_hbm.at[idx_slice], bufs[q], gsem).wait()

            _zero_rows(bufs[q], zc, prev)

            pltpu.async_copy(
                bufs[q], out_hbm.at[pl.ds(row0 + g * C, C)], wsems[q])
        return 0

    lax.fori_loop(0, NCH // 2, chunk_pair, 0)

    for q in range(2):
        g = 2 * (NCH - 2 + q) + par
        pltpu.make_async_copy(
            bufs[q], out_hbm.at[pl.ds(row0 + g * C, C)], wsems[q]
        ).wait()


@jax.jit
def _length_regulate(x_flat, duration):
    mesh = plsc.VectorSubcoreMesh(core_axis_name="c", subcore_axis_name="s")
    out, mel = pl.kernel(
        _lr_body,
        out_type=[
            jax.ShapeDtypeStruct((B * T, D), jnp.float32),
            jax.ShapeDtypeStruct((B,), jnp.int32),
        ],
        mesh=mesh,
        compiler_params=pltpu.CompilerParams(needs_layout_passes=False),
        scratch_types=[
            pltpu.VMEM((P,), jnp.int32),
            pltpu.VMEM((T,), jnp.int32),
            pltpu.VMEM((T,), jnp.int32),
            pltpu.VMEM((2, C, D), jnp.float32),
            pltpu.VMEM((L,), jnp.int32),
            pltpu.VMEM((B, L), jnp.int32),
            pltpu.VMEM_SHARED((B, L), jnp.int32),
            pltpu.SemaphoreType.DMA,
            pltpu.SemaphoreType.DMA,
            pltpu.SemaphoreType.DMA,
        ],
    )(x_flat, duration)
    return out, mel


def kernel(x, duration, max_len):
    x_flat = x.reshape(B * P, D)
    out, mel = _length_regulate(x_flat, duration.astype(jnp.int32))
    return out.reshape(B, T, D), mel
